# baseline (speedup 1.0000x reference)
import jax
from jax.experimental import pallas as pl
from jax.experimental.pallas import tpu as pltpu

_GRID = 25


def _copy_body(n_in, ei_in, e_in, u_in, b_in,
               n_out, ei_out, e_out, u_out, b_out,
               u_sem, b_sem):
    i = pl.program_id(0)

    @pl.when(i == 0)
    def _start_small():
        pltpu.make_async_copy(u_in, u_out, u_sem).start()
        pltpu.make_async_copy(b_in, b_out, b_sem).start()

    n_out[...] = n_in[...]
    ei_out[...] = ei_in[...]
    e_out[...] = e_in[...]

    @pl.when(i == pl.num_programs(0) - 1)
    def _wait_small():
        pltpu.make_async_copy(u_in, u_out, u_sem).wait()
        pltpu.make_async_copy(b_in, b_out, b_sem).wait()


def kernel(nodes, edge_index, edges, u, batch):
    g = _GRID
    nf = nodes.reshape(-1)
    eif = edge_index.reshape(-1)
    ef = edges.reshape(-1)
    any_spec = pl.BlockSpec(memory_space=pl.ANY)
    specs = [
        pl.BlockSpec((nf.shape[0] // g,), lambda i: (i,)),
        pl.BlockSpec((eif.shape[0] // g,), lambda i: (i,)),
        pl.BlockSpec((ef.shape[0] // g,), lambda i: (i,)),
        any_spec,
        any_spec,
    ]
    out = pl.pallas_call(
        _copy_body,
        grid=(g,),
        in_specs=specs,
        out_specs=specs,
        out_shape=[
            jax.ShapeDtypeStruct(nf.shape, nf.dtype),
            jax.ShapeDtypeStruct(eif.shape, eif.dtype),
            jax.ShapeDtypeStruct(ef.shape, ef.dtype),
            jax.ShapeDtypeStruct(u.shape, u.dtype),
            jax.ShapeDtypeStruct(batch.shape, batch.dtype),
        ],
        scratch_shapes=[pltpu.SemaphoreType.DMA, pltpu.SemaphoreType.DMA],
    )(nf, eif, ef, u, batch)
    return (out[0].reshape(nodes.shape), out[1].reshape(edge_index.shape),
            out[2].reshape(edges.shape), out[3], out[4])


# speedup vs baseline: 1.0252x; 1.0252x over previous
import functools

import jax
from jax import lax
from jax.experimental import pallas as pl
from jax.experimental.pallas import tpu as pltpu
from jax.experimental.pallas import tpu_sc as plsc

_TC_GRID = 10
_E_ROWS = 320000
_E_COLS = 16
_NW = 32
_ROWS_PER_W = _E_ROWS // _NW
_CHUNK = 2000
_NCHUNK = _ROWS_PER_W // _CHUNK


def _tc_body(n_in, ei_in, u_in, b_in, n_out, ei_out, u_out, b_out,
             u_sem, b_sem):
    i = pl.program_id(0)

    @pl.when(i == 0)
    def _start_small():
        pltpu.make_async_copy(u_in, u_out, u_sem).start()
        pltpu.make_async_copy(b_in, b_out, b_sem).start()

    n_out[...] = n_in[...]
    ei_out[...] = ei_in[...]

    @pl.when(i == pl.num_programs(0) - 1)
    def _wait_small():
        pltpu.make_async_copy(u_in, u_out, u_sem).wait()
        pltpu.make_async_copy(b_in, b_out, b_sem).wait()


def _sc_body(e_in, e_out, buf, sem):
    wid = lax.axis_index("s") * 2 + lax.axis_index("c")
    base = wid * _ROWS_PER_W
    for k in range(_NCHUNK):
        off = base + k * _CHUNK
        pltpu.sync_copy(e_in.at[pl.ds(off, _CHUNK)], buf)
        pltpu.sync_copy(buf, e_out.at[pl.ds(off, _CHUNK)])


def _sc_edges_copy(edges):
    mesh = plsc.VectorSubcoreMesh(core_axis_name="c", subcore_axis_name="s")
    return pl.kernel(
        _sc_body,
        out_type=jax.ShapeDtypeStruct(edges.shape, edges.dtype),
        mesh=mesh,
        scratch_types=[
            pltpu.VMEM((_CHUNK, _E_COLS), edges.dtype),
            pltpu.SemaphoreType.DMA,
        ],
        compiler_params=pltpu.CompilerParams(use_tc_tiling_on_sc=False),
    )(edges)


def kernel(nodes, edge_index, edges, u, batch):
    g = _TC_GRID
    any_spec = pl.BlockSpec(memory_space=pl.ANY)
    specs = [
        pl.BlockSpec((nodes.shape[0] // g, nodes.shape[1]), lambda i: (i, 0)),
        pl.BlockSpec((edge_index.shape[0], edge_index.shape[1] // g),
                     lambda i: (0, i)),
        any_spec,
        any_spec,
    ]
    e_out = _sc_edges_copy(edges)
    out = pl.pallas_call(
        _tc_body,
        grid=(g,),
        in_specs=specs,
        out_specs=specs,
        out_shape=[
            jax.ShapeDtypeStruct(nodes.shape, nodes.dtype),
            jax.ShapeDtypeStruct(edge_index.shape, edge_index.dtype),
            jax.ShapeDtypeStruct(u.shape, u.dtype),
            jax.ShapeDtypeStruct(batch.shape, batch.dtype),
        ],
        scratch_shapes=[pltpu.SemaphoreType.DMA, pltpu.SemaphoreType.DMA],
    )(nodes, edge_index, u, batch)
    return (out[0], out[1], e_out, out[2], out[3])


# speedup vs baseline: 1.1388x; 1.1108x over previous
import jax
from jax.experimental import pallas as pl
from jax.experimental.pallas import tpu as pltpu

_GRID = 25


def _copy_body(n_in, ei_in, e_in, u_in, b_in,
               n_out, ei_out, e_out, u_out, b_out,
               u_sem, b_sem):
    i = pl.program_id(0)

    @pl.when(i == 0)
    def _start_small():
        pltpu.make_async_copy(u_in, u_out, u_sem).start()
        pltpu.make_async_copy(b_in, b_out, b_sem).start()

    n_out[...] = n_in[...]
    ei_out[...] = ei_in[...]
    e_out[...] = e_in[...]

    @pl.when(i == pl.num_programs(0) - 1)
    def _wait_small():
        pltpu.make_async_copy(u_in, u_out, u_sem).wait()
        pltpu.make_async_copy(b_in, b_out, b_sem).wait()


def kernel(nodes, edge_index, edges, u, batch):
    g = _GRID
    any_spec = pl.BlockSpec(memory_space=pl.ANY)
    specs = [
        pl.BlockSpec((nodes.shape[0] // g, nodes.shape[1]), lambda i: (i, 0)),
        pl.BlockSpec((edge_index.shape[0], edge_index.shape[1] // g),
                     lambda i: (0, i)),
        pl.BlockSpec((edges.shape[0] // g, edges.shape[1]), lambda i: (i, 0)),
        any_spec,
        any_spec,
    ]
    out = pl.pallas_call(
        _copy_body,
        grid=(g,),
        in_specs=specs,
        out_specs=specs,
        out_shape=[
            jax.ShapeDtypeStruct(nodes.shape, nodes.dtype),
            jax.ShapeDtypeStruct(edge_index.shape, edge_index.dtype),
            jax.ShapeDtypeStruct(edges.shape, edges.dtype),
            jax.ShapeDtypeStruct(u.shape, u.dtype),
            jax.ShapeDtypeStruct(batch.shape, batch.dtype),
        ],
        scratch_shapes=[pltpu.SemaphoreType.DMA, pltpu.SemaphoreType.DMA],
    )(nodes, edge_index, edges, u, batch)
    return (out[0], out[1], out[2], out[3], out[4])
